# baseline (speedup 1.0000x reference)
import functools

import jax
import jax.numpy as jnp
from jax import lax
from jax.experimental import pallas as pl
from jax.experimental.pallas import tpu as pltpu
from jax.experimental.pallas import tpu_sc as plsc

H = 256
N_LEAVES = 16384
DEPTH = 8
LEVEL_SIZES = [N_LEAVES >> l for l in range(DEPTH)]
N_TOTAL = sum(LEVEL_SIZES)
LEAF_BLK = 2048
LVL_BLK = 2048


def _sc_gather(table, idx):
    B = idx.shape[0]
    D = table.shape[1]
    NW = 32
    b_per_w = B // NW
    CH = 128
    n_ch = b_per_w // CH

    mesh = plsc.VectorSubcoreMesh(core_axis_name="c", subcore_axis_name="s")

    @functools.partial(
        pl.kernel, mesh=mesh,
        out_type=jax.ShapeDtypeStruct((B, D), jnp.float32),
        scratch_types=[
            pltpu.VMEM((b_per_w,), jnp.int32),
            pltpu.VMEM((CH, D), jnp.float32),
            pltpu.VMEM((CH, D), jnp.float32),
            pltpu.SemaphoreType.DMA,
            pltpu.SemaphoreType.DMA,
            pltpu.SemaphoreType.DMA,
            pltpu.SemaphoreType.DMA,
        ],
    )
    def k(table_hbm, idx_hbm, out_hbm, idx_v, r0, r1, sg0, sg1, so0, so1):
        wid = lax.axis_index("s") * 2 + lax.axis_index("c")
        base = wid * b_per_w
        pltpu.sync_copy(idx_hbm.at[pl.ds(base, b_per_w)], idx_v)
        rows = (r0, r1)
        sg = (sg0, sg1)
        so = (so0, so1)

        def gather(ch):
            return pltpu.async_copy(
                table_hbm.at[idx_v.at[pl.ds(ch * CH, CH)]],
                rows[ch % 2], sg[ch % 2])

        def put(ch):
            return pltpu.async_copy(
                rows[ch % 2], out_hbm.at[pl.ds(base + ch * CH, CH)],
                so[ch % 2])

        g_cur = gather(0)
        outs = [None] * n_ch
        for ch in range(n_ch):
            if ch + 1 < n_ch:
                if ch >= 1:
                    outs[ch - 1].wait()
                g_next = gather(ch + 1)
            g_cur.wait()
            outs[ch] = put(ch)
            if ch + 1 < n_ch:
                g_cur = g_next
        outs[n_ch - 2].wait()
        outs[n_ch - 1].wait()

    return k(table, idx)


def _sig(x):
    return 0.5 * jnp.tanh(0.5 * x) + 0.5


def _tree_body(emb_hbm, wiou_ref, biou_ref, ucat_ref, bcat_ref, lw2_ref,
               lb2_ref, lw_ref, lb_ref, lgp_ref, lg7_ref,
               ebuf, hA, hB, cA, cB, sems):
    def emb_copy(i, slot):
        return pltpu.make_async_copy(
            emb_hbm.at[pl.ds(pl.multiple_of(i * LEAF_BLK, LEAF_BLK),
                             LEAF_BLK), :],
            ebuf.at[slot], sems.at[slot])

    def leaf_block(i, slot):
        emb_copy(i, slot).wait()
        e = ebuf[slot].astype(jnp.bfloat16)
        zi = jnp.dot(e, wiou_ref[:, :H],
                     preferred_element_type=jnp.float32) + biou_ref[:, :H]
        zu = jnp.dot(e, wiou_ref[:, 2 * H:],
                     preferred_element_type=jnp.float32) + biou_ref[:, 2 * H:]
        ig = _sig(zi)
        ug = jnp.tanh(zu)
        zo = jnp.dot(e, wiou_ref[:, H:2 * H],
                     preferred_element_type=jnp.float32) + biou_ref[:, H:2 * H]
        c0 = ig * ug
        og = _sig(zo)
        h0 = (og * jnp.tanh(c0)).astype(jnp.bfloat16)
        half = pl.multiple_of(i * (LEAF_BLK // 2), LEAF_BLK // 2)
        hA[pl.ds(half, LEAF_BLK // 2), :] = h0.reshape(LEAF_BLK // 2, 2 * H)
        cA[pl.ds(half, LEAF_BLK // 2), :] = (
            c0.astype(jnp.bfloat16).reshape(LEAF_BLK // 2, 2 * H))

    n_leaf_blk = N_LEAVES // LEAF_BLK
    emb_copy(0, 0).start()

    def leaf_pair(p, carry):
        i0 = 2 * p
        emb_copy(i0 + 1, 1).start()
        leaf_block(i0, 0)

        @pl.when(i0 + 2 < n_leaf_blk)
        def _():
            emb_copy(i0 + 2, 0).start()
        leaf_block(i0 + 1, 1)
        return carry

    lax.fori_loop(0, n_leaf_blk // 2, leaf_pair, 0)

    def level_block(h_in, c_in, h_out, c_out, blk, jb, poff, last):
        jb = pl.multiple_of(jb, blk)
        h2 = h_in[pl.ds(jb, blk), :]
        c2 = c_in[pl.ds(jb, blk), :]
        lgp_ref[pl.ds(pl.multiple_of(poff + jb, 8), blk), :] = jnp.dot(
            h2, lw2_ref[...], preferred_element_type=jnp.float32) + lb2_ref[...]
        zf = jnp.dot(h2, ucat_ref[:, :2 * H],
                     preferred_element_type=jnp.float32) + bcat_ref[:, :2 * H]
        zi = jnp.dot(h2, ucat_ref[:, 2 * H:3 * H],
                     preferred_element_type=jnp.float32) + bcat_ref[:, 2 * H:3 * H]
        f = _sig(zf)
        cf = f[:, :H] * c2[:, :H] + f[:, H:] * c2[:, H:]
        zu = jnp.dot(h2, ucat_ref[:, 4 * H:],
                     preferred_element_type=jnp.float32) + bcat_ref[:, 4 * H:]
        ig = _sig(zi)
        zo = jnp.dot(h2, ucat_ref[:, 3 * H:4 * H],
                     preferred_element_type=jnp.float32) + bcat_ref[:, 3 * H:4 * H]
        ug = jnp.tanh(zu)
        og = _sig(zo)
        c_new = ig * ug + cf
        h_new = (og * jnp.tanh(c_new)).astype(jnp.bfloat16)
        if not last:
            jh = pl.multiple_of(jb // 2, blk // 2)
            h_out[pl.ds(jh, blk // 2), :] = h_new.reshape(blk // 2, 2 * H)
            c_out[pl.ds(jh, blk // 2), :] = (
                c_new.astype(jnp.bfloat16).reshape(blk // 2, 2 * H))
        if last:
            lg7_ref[...] = jnp.dot(
                h_new, lw_ref[...],
                preferred_element_type=jnp.float32) + lb_ref[...]

    poff = 0
    bufs = [(hA, cA), (hB, cB)]
    for l in range(1, DEPTH):
        n_l = LEVEL_SIZES[l]
        h_in, c_in = bufs[(l - 1) % 2]
        h_out, c_out = bufs[l % 2]
        blk = min(n_l, LVL_BLK)
        n_blk = n_l // blk
        last = l == DEPTH - 1
        off_l = poff
        if n_blk > 2:
            def level_step(j, carry, h_in=h_in, c_in=c_in, h_out=h_out,
                           c_out=c_out, blk=blk, off_l=off_l, last=last):
                level_block(h_in, c_in, h_out, c_out, blk, 2 * j * blk,
                            off_l, last)
                level_block(h_in, c_in, h_out, c_out, blk,
                            (2 * j + 1) * blk, off_l, last)
                return carry
            lax.fori_loop(0, n_blk // 2, level_step, 0)
        elif n_blk == 2:
            level_block(h_in, c_in, h_out, c_out, blk, 0, off_l, last)
            level_block(h_in, c_in, h_out, c_out, blk, blk, off_l, last)
        else:
            level_block(h_in, c_in, h_out, c_out, blk, 0, off_l, last)
        poff += n_l


def _tree(embeds, W_iou_w, W_iou_b, U_cat, b_cat, lin2_w, lin2_b,
          lin_w, lin_b):
    NC = lin_w.shape[1]
    n_pairs = (N_TOTAL - LEVEL_SIZES[-1]) // 2
    vmem = pl.BlockSpec(memory_space=pltpu.MemorySpace.VMEM)
    return pl.pallas_call(
        _tree_body,
        in_specs=[
            pl.BlockSpec(memory_space=pltpu.MemorySpace.HBM),
            vmem, vmem, vmem, vmem, vmem, vmem, vmem, vmem,
        ],
        out_specs=[vmem, vmem],
        out_shape=[
            jax.ShapeDtypeStruct((n_pairs, 2 * NC), jnp.float32),
            jax.ShapeDtypeStruct((LEVEL_SIZES[-1], NC), jnp.float32),
        ],
        scratch_shapes=[
            pltpu.VMEM((2, LEAF_BLK, H), jnp.float32),
            pltpu.VMEM((N_LEAVES // 2, 2 * H), jnp.bfloat16),
            pltpu.VMEM((N_LEAVES // 4, 2 * H), jnp.bfloat16),
            pltpu.VMEM((N_LEAVES // 2, 2 * H), jnp.bfloat16),
            pltpu.VMEM((N_LEAVES // 4, 2 * H), jnp.bfloat16),
            pltpu.SemaphoreType.DMA((2,)),
        ],
    )(embeds, W_iou_w, W_iou_b, U_cat, b_cat, lin2_w, lin2_b, lin_w, lin_b)


def kernel(wordid, x, h, c, emb, W_iou_w, W_iou_b, U_iou_w, U_iou_b,
           U_f_w, U_f_b, lin_w, lin_b):
    del x, h, c

    embeds = _sc_gather(emb, wordid.astype(jnp.int32))

    U_cat = jnp.concatenate([U_f_w, U_iou_w],
                            axis=1).astype(jnp.bfloat16)
    b_cat = jnp.concatenate([U_f_b, U_iou_b]).reshape(1, -1)

    NC = lin_w.shape[1]
    zero = jnp.zeros_like(lin_w)
    lin2_w = jnp.concatenate([
        jnp.concatenate([lin_w, zero], axis=1),
        jnp.concatenate([zero, lin_w], axis=1)], axis=0)
    lin2_b = jnp.concatenate([lin_b, lin_b]).reshape(1, -1)

    lgp, lg7 = _tree(embeds,
                     W_iou_w.astype(jnp.bfloat16), W_iou_b.reshape(1, -1),
                     U_cat, b_cat,
                     lin2_w.astype(jnp.bfloat16), lin2_b,
                     lin_w.astype(jnp.bfloat16), lin_b.reshape(1, -1))
    return jnp.concatenate([lgp.reshape(-1, NC), lg7], axis=0)


# speedup vs baseline: 1.0100x; 1.0100x over previous
import functools

import jax
import jax.numpy as jnp
from jax import lax
from jax.experimental import pallas as pl
from jax.experimental.pallas import tpu as pltpu
from jax.experimental.pallas import tpu_sc as plsc

H = 256
N_LEAVES = 16384
DEPTH = 8
LEVEL_SIZES = [N_LEAVES >> l for l in range(DEPTH)]
N_TOTAL = sum(LEVEL_SIZES)
LEAF_BLK = 2048
LVL_BLK = 2048


def _sc_gather(table, idx):
    B = idx.shape[0]
    D = table.shape[1]
    NW = 32
    b_per_w = B // NW
    CH = 128
    n_ch = b_per_w // CH

    mesh = plsc.VectorSubcoreMesh(core_axis_name="c", subcore_axis_name="s")

    @functools.partial(
        pl.kernel, mesh=mesh,
        out_type=jax.ShapeDtypeStruct((B, D), jnp.float32),
        scratch_types=[
            pltpu.VMEM((b_per_w,), jnp.int32),
            pltpu.VMEM((CH, D), jnp.float32),
            pltpu.VMEM((CH, D), jnp.float32),
            pltpu.SemaphoreType.DMA,
            pltpu.SemaphoreType.DMA,
            pltpu.SemaphoreType.DMA,
            pltpu.SemaphoreType.DMA,
        ],
    )
    def k(table_hbm, idx_hbm, out_hbm, idx_v, r0, r1, sg0, sg1, so0, so1):
        wid = lax.axis_index("s") * 2 + lax.axis_index("c")
        base = wid * b_per_w
        pltpu.sync_copy(idx_hbm.at[pl.ds(base, b_per_w)], idx_v)
        rows = (r0, r1)
        sg = (sg0, sg1)
        so = (so0, so1)

        def gather(ch):
            return pltpu.async_copy(
                table_hbm.at[idx_v.at[pl.ds(ch * CH, CH)]],
                rows[ch % 2], sg[ch % 2])

        def put(ch):
            return pltpu.async_copy(
                rows[ch % 2], out_hbm.at[pl.ds(base + ch * CH, CH)],
                so[ch % 2])

        g_cur = gather(0)
        outs = [None] * n_ch
        for ch in range(n_ch):
            if ch + 1 < n_ch:
                if ch >= 1:
                    outs[ch - 1].wait()
                g_next = gather(ch + 1)
            g_cur.wait()
            outs[ch] = put(ch)
            if ch + 1 < n_ch:
                g_cur = g_next
        outs[n_ch - 2].wait()
        outs[n_ch - 1].wait()

    return k(table, idx)


def _sig(x):
    return 0.5 * jnp.tanh(0.5 * x) + 0.5


def _tree_body(emb_hbm, wiou_ref, biou_ref, ucat_ref, bcat_ref, lw2_ref,
               lb2_ref, lw_ref, lb_ref, lgp_ref, lg7_ref,
               ebuf, hA, hB, cA, cB, sems):
    def emb_copy(i, slot):
        return pltpu.make_async_copy(
            emb_hbm.at[pl.ds(pl.multiple_of(i * LEAF_BLK, LEAF_BLK),
                             LEAF_BLK), :],
            ebuf.at[slot], sems.at[slot])

    def leaf_block(i, slot):
        emb_copy(i, slot).wait()
        e = ebuf[slot].astype(jnp.bfloat16)
        zi = jnp.dot(e, wiou_ref[:, :H],
                     preferred_element_type=jnp.float32) + biou_ref[:, :H]
        zu = jnp.dot(e, wiou_ref[:, 2 * H:],
                     preferred_element_type=jnp.float32) + biou_ref[:, 2 * H:]
        ig = _sig(zi)
        ug = jnp.tanh(zu)
        zo = jnp.dot(e, wiou_ref[:, H:2 * H],
                     preferred_element_type=jnp.float32) + biou_ref[:, H:2 * H]
        c0 = ig * ug
        og = _sig(zo)
        h0 = (og * jnp.tanh(c0)).astype(jnp.bfloat16)
        half = pl.multiple_of(i * (LEAF_BLK // 2), LEAF_BLK // 2)
        hA[pl.ds(half, LEAF_BLK // 2), :] = h0.reshape(LEAF_BLK // 2, 2 * H)
        cA[pl.ds(half, LEAF_BLK // 2), :] = (
            c0.astype(jnp.bfloat16).reshape(LEAF_BLK // 2, 2 * H))

    n_leaf_blk = N_LEAVES // LEAF_BLK
    emb_copy(0, 0).start()

    def leaf_pair(p, carry):
        i0 = 2 * p
        emb_copy(i0 + 1, 1).start()
        leaf_block(i0, 0)

        @pl.when(i0 + 2 < n_leaf_blk)
        def _():
            emb_copy(i0 + 2, 0).start()
        leaf_block(i0 + 1, 1)
        return carry

    lax.fori_loop(0, n_leaf_blk // 2, leaf_pair, 0)

    def level_block(h_in, c_in, h_out, c_out, blk, jb, poff, last):
        jb = pl.multiple_of(jb, blk)
        h2 = h_in[pl.ds(jb, blk), :]
        c2 = c_in[pl.ds(jb, blk), :]
        zf = jnp.dot(h2, ucat_ref[:, :2 * H],
                     preferred_element_type=jnp.float32) + bcat_ref[:, :2 * H]
        zi = jnp.dot(h2, ucat_ref[:, 2 * H:3 * H],
                     preferred_element_type=jnp.float32) + bcat_ref[:, 2 * H:3 * H]
        f = _sig(zf)
        cf = f[:, :H] * c2[:, :H] + f[:, H:] * c2[:, H:]
        zu = jnp.dot(h2, ucat_ref[:, 4 * H:],
                     preferred_element_type=jnp.float32) + bcat_ref[:, 4 * H:]
        ig = _sig(zi)
        zo = jnp.dot(h2, ucat_ref[:, 3 * H:4 * H],
                     preferred_element_type=jnp.float32) + bcat_ref[:, 3 * H:4 * H]
        ug = jnp.tanh(zu)
        og = _sig(zo)
        lgp_ref[pl.ds(pl.multiple_of(poff + jb, 8), blk), :] = jnp.dot(
            h2, lw2_ref[...], preferred_element_type=jnp.float32) + lb2_ref[...]
        c_new = ig * ug + cf
        h_new = (og * jnp.tanh(c_new)).astype(jnp.bfloat16)
        if not last:
            jh = pl.multiple_of(jb // 2, blk // 2)
            h_out[pl.ds(jh, blk // 2), :] = h_new.reshape(blk // 2, 2 * H)
            c_out[pl.ds(jh, blk // 2), :] = (
                c_new.astype(jnp.bfloat16).reshape(blk // 2, 2 * H))
        if last:
            lg7_ref[...] = jnp.dot(
                h_new, lw_ref[...],
                preferred_element_type=jnp.float32) + lb_ref[...]

    poff = 0
    bufs = [(hA, cA), (hB, cB)]
    for l in range(1, DEPTH):
        n_l = LEVEL_SIZES[l]
        h_in, c_in = bufs[(l - 1) % 2]
        h_out, c_out = bufs[l % 2]
        blk = min(n_l, LVL_BLK)
        n_blk = n_l // blk
        last = l == DEPTH - 1
        off_l = poff
        if n_blk > 2:
            def level_step(j, carry, h_in=h_in, c_in=c_in, h_out=h_out,
                           c_out=c_out, blk=blk, off_l=off_l, last=last):
                level_block(h_in, c_in, h_out, c_out, blk, 2 * j * blk,
                            off_l, last)
                level_block(h_in, c_in, h_out, c_out, blk,
                            (2 * j + 1) * blk, off_l, last)
                return carry
            lax.fori_loop(0, n_blk // 2, level_step, 0)
        elif n_blk == 2:
            level_block(h_in, c_in, h_out, c_out, blk, 0, off_l, last)
            level_block(h_in, c_in, h_out, c_out, blk, blk, off_l, last)
        else:
            level_block(h_in, c_in, h_out, c_out, blk, 0, off_l, last)
        poff += n_l


def _tree(embeds, W_iou_w, W_iou_b, U_cat, b_cat, lin2_w, lin2_b,
          lin_w, lin_b):
    NC = lin_w.shape[1]
    n_pairs = (N_TOTAL - LEVEL_SIZES[-1]) // 2
    vmem = pl.BlockSpec(memory_space=pltpu.MemorySpace.VMEM)
    return pl.pallas_call(
        _tree_body,
        in_specs=[
            pl.BlockSpec(memory_space=pltpu.MemorySpace.HBM),
            vmem, vmem, vmem, vmem, vmem, vmem, vmem, vmem,
        ],
        out_specs=[vmem, vmem],
        out_shape=[
            jax.ShapeDtypeStruct((n_pairs, 2 * NC), jnp.float32),
            jax.ShapeDtypeStruct((LEVEL_SIZES[-1], NC), jnp.float32),
        ],
        scratch_shapes=[
            pltpu.VMEM((2, LEAF_BLK, H), jnp.float32),
            pltpu.VMEM((N_LEAVES // 2, 2 * H), jnp.bfloat16),
            pltpu.VMEM((N_LEAVES // 4, 2 * H), jnp.bfloat16),
            pltpu.VMEM((N_LEAVES // 2, 2 * H), jnp.bfloat16),
            pltpu.VMEM((N_LEAVES // 4, 2 * H), jnp.bfloat16),
            pltpu.SemaphoreType.DMA((2,)),
        ],
    )(embeds, W_iou_w, W_iou_b, U_cat, b_cat, lin2_w, lin2_b, lin_w, lin_b)


def kernel(wordid, x, h, c, emb, W_iou_w, W_iou_b, U_iou_w, U_iou_b,
           U_f_w, U_f_b, lin_w, lin_b):
    del x, h, c

    embeds = _sc_gather(emb, wordid.astype(jnp.int32))

    U_cat = jnp.concatenate([U_f_w, U_iou_w],
                            axis=1).astype(jnp.bfloat16)
    b_cat = jnp.concatenate([U_f_b, U_iou_b]).reshape(1, -1)

    NC = lin_w.shape[1]
    zero = jnp.zeros_like(lin_w)
    lin2_w = jnp.concatenate([
        jnp.concatenate([lin_w, zero], axis=1),
        jnp.concatenate([zero, lin_w], axis=1)], axis=0)
    lin2_b = jnp.concatenate([lin_b, lin_b]).reshape(1, -1)

    lgp, lg7 = _tree(embeds,
                     W_iou_w.astype(jnp.bfloat16), W_iou_b.reshape(1, -1),
                     U_cat, b_cat,
                     lin2_w.astype(jnp.bfloat16), lin2_b,
                     lin_w.astype(jnp.bfloat16), lin_b.reshape(1, -1))
    return jnp.concatenate([lgp.reshape(-1, NC), lg7], axis=0)


# speedup vs baseline: 1.1183x; 1.1072x over previous
import functools

import jax
import jax.numpy as jnp
from jax import lax
from jax.experimental import pallas as pl
from jax.experimental.pallas import tpu as pltpu
from jax.experimental.pallas import tpu_sc as plsc

H = 256
N_LEAVES = 16384
DEPTH = 8
LEVEL_SIZES = [N_LEAVES >> l for l in range(DEPTH)]
N_TOTAL = sum(LEVEL_SIZES)
LEAF_BLK = 2048
LVL_BLK = 2048


def _sc_gather(table, idx):
    B = idx.shape[0]
    D = table.shape[1]
    NW = 32
    b_per_w = B // NW
    CH = 128
    n_ch = b_per_w // CH

    mesh = plsc.VectorSubcoreMesh(core_axis_name="c", subcore_axis_name="s")

    @functools.partial(
        pl.kernel, mesh=mesh,
        out_type=jax.ShapeDtypeStruct((B, D), jnp.float32),
        scratch_types=[
            pltpu.VMEM((b_per_w,), jnp.int32),
            pltpu.VMEM((CH, D), jnp.float32),
            pltpu.VMEM((CH, D), jnp.float32),
            pltpu.SemaphoreType.DMA,
            pltpu.SemaphoreType.DMA,
            pltpu.SemaphoreType.DMA,
            pltpu.SemaphoreType.DMA,
        ],
    )
    def k(table_hbm, idx_hbm, out_hbm, idx_v, r0, r1, sg0, sg1, so0, so1):
        wid = lax.axis_index("s") * 2 + lax.axis_index("c")
        base = wid * b_per_w
        pltpu.sync_copy(idx_hbm.at[pl.ds(base, b_per_w)], idx_v)
        rows = (r0, r1)
        sg = (sg0, sg1)
        so = (so0, so1)

        def gather(ch):
            return pltpu.async_copy(
                table_hbm.at[idx_v.at[pl.ds(ch * CH, CH)]],
                rows[ch % 2], sg[ch % 2])

        def put(ch):
            return pltpu.async_copy(
                rows[ch % 2], out_hbm.at[pl.ds(base + ch * CH, CH)],
                so[ch % 2])

        g_cur = gather(0)
        outs = [None] * n_ch
        for ch in range(n_ch):
            if ch + 1 < n_ch:
                if ch >= 1:
                    outs[ch - 1].wait()
                g_next = gather(ch + 1)
            g_cur.wait()
            outs[ch] = put(ch)
            if ch + 1 < n_ch:
                g_cur = g_next
        outs[n_ch - 2].wait()
        outs[n_ch - 1].wait()

    return k(table, idx)


def _sig(x):
    return 0.5 * jnp.tanh(0.5 * x) + 0.5


def _tree_body(emb_hbm, wiou_ref, biou_ref, ucat_ref, bcat_ref, lw_ref,
               lb_ref, lg_ref, ebuf, hA, hB, cA, cB, sems):
    def emb_copy(i, slot):
        return pltpu.make_async_copy(
            emb_hbm.at[pl.ds(pl.multiple_of(i * LEAF_BLK, LEAF_BLK),
                             LEAF_BLK), :],
            ebuf.at[slot], sems.at[slot])

    def leaf_block(i, slot):
        emb_copy(i, slot).wait()
        e = ebuf[slot].astype(jnp.bfloat16)
        zi = jnp.dot(e, wiou_ref[:, :H],
                     preferred_element_type=jnp.float32) + biou_ref[:, :H]
        zu = jnp.dot(e, wiou_ref[:, 2 * H:],
                     preferred_element_type=jnp.float32) + biou_ref[:, 2 * H:]
        ig = _sig(zi)
        ug = jnp.tanh(zu)
        zo = jnp.dot(e, wiou_ref[:, H:2 * H],
                     preferred_element_type=jnp.float32) + biou_ref[:, H:2 * H]
        c0 = ig * ug
        og = _sig(zo)
        h0 = (og * jnp.tanh(c0)).astype(jnp.bfloat16)
        half = pl.multiple_of(i * (LEAF_BLK // 2), LEAF_BLK // 2)
        hA[pl.ds(half, LEAF_BLK // 2), :] = h0.reshape(LEAF_BLK // 2, 2 * H)
        cA[pl.ds(half, LEAF_BLK // 2), :] = (
            c0.astype(jnp.bfloat16).reshape(LEAF_BLK // 2, 2 * H))
        lg_ref[pl.ds(pl.multiple_of(i * LEAF_BLK, LEAF_BLK),
                     LEAF_BLK), :] = jnp.dot(
            h0, lw_ref[...], preferred_element_type=jnp.float32) + lb_ref[...]

    n_leaf_blk = N_LEAVES // LEAF_BLK
    emb_copy(0, 0).start()

    def leaf_pair(p, carry):
        i0 = 2 * p
        emb_copy(i0 + 1, 1).start()
        leaf_block(i0, 0)

        @pl.when(i0 + 2 < n_leaf_blk)
        def _():
            emb_copy(i0 + 2, 0).start()
        leaf_block(i0 + 1, 1)
        return carry

    lax.fori_loop(0, n_leaf_blk // 2, leaf_pair, 0)

    def level_block(h_in, c_in, h_out, c_out, blk, jb, off, last):
        jb = pl.multiple_of(jb, blk)
        h2 = h_in[pl.ds(jb, blk), :]
        c2 = c_in[pl.ds(jb, blk), :]
        zf = jnp.dot(h2, ucat_ref[:, :2 * H],
                     preferred_element_type=jnp.float32) + bcat_ref[:, :2 * H]
        zi = jnp.dot(h2, ucat_ref[:, 2 * H:3 * H],
                     preferred_element_type=jnp.float32) + bcat_ref[:, 2 * H:3 * H]
        f = _sig(zf)
        cf = f[:, :H] * c2[:, :H] + f[:, H:] * c2[:, H:]
        zu = jnp.dot(h2, ucat_ref[:, 4 * H:],
                     preferred_element_type=jnp.float32) + bcat_ref[:, 4 * H:]
        ig = _sig(zi)
        zo = jnp.dot(h2, ucat_ref[:, 3 * H:4 * H],
                     preferred_element_type=jnp.float32) + bcat_ref[:, 3 * H:4 * H]
        ug = jnp.tanh(zu)
        og = _sig(zo)
        c_new = ig * ug + cf
        h_new = (og * jnp.tanh(c_new)).astype(jnp.bfloat16)
        if not last:
            jh = pl.multiple_of(jb // 2, blk // 2)
            h_out[pl.ds(jh, blk // 2), :] = h_new.reshape(blk // 2, 2 * H)
            c_out[pl.ds(jh, blk // 2), :] = (
                c_new.astype(jnp.bfloat16).reshape(blk // 2, 2 * H))
        lg_ref[pl.ds(pl.multiple_of(off + jb, blk), blk), :] = jnp.dot(
            h_new, lw_ref[...],
            preferred_element_type=jnp.float32) + lb_ref[...]

    off = N_LEAVES
    bufs = [(hA, cA), (hB, cB)]
    for l in range(1, DEPTH):
        n_l = LEVEL_SIZES[l]
        h_in, c_in = bufs[(l - 1) % 2]
        h_out, c_out = bufs[l % 2]
        blk = min(n_l, LVL_BLK)
        n_blk = n_l // blk
        last = l == DEPTH - 1
        off_l = off
        if n_blk > 2:
            def level_step(j, carry, h_in=h_in, c_in=c_in, h_out=h_out,
                           c_out=c_out, blk=blk, off_l=off_l, last=last):
                level_block(h_in, c_in, h_out, c_out, blk, 2 * j * blk,
                            off_l, last)
                level_block(h_in, c_in, h_out, c_out, blk,
                            (2 * j + 1) * blk, off_l, last)
                return carry
            lax.fori_loop(0, n_blk // 2, level_step, 0)
        elif n_blk == 2:
            level_block(h_in, c_in, h_out, c_out, blk, 0, off_l, last)
            level_block(h_in, c_in, h_out, c_out, blk, blk, off_l, last)
        else:
            level_block(h_in, c_in, h_out, c_out, blk, 0, off_l, last)
        off += n_l


def _tree(embeds, W_iou_w, W_iou_b, U_cat, b_cat, lin_w, lin_b):
    NC = lin_w.shape[1]
    vmem = pl.BlockSpec(memory_space=pltpu.MemorySpace.VMEM)
    return pl.pallas_call(
        _tree_body,
        in_specs=[
            pl.BlockSpec(memory_space=pltpu.MemorySpace.HBM),
            vmem, vmem, vmem, vmem, vmem, vmem,
        ],
        out_specs=pl.BlockSpec(memory_space=pltpu.MemorySpace.VMEM),
        out_shape=jax.ShapeDtypeStruct((N_TOTAL, NC), jnp.float32),
        scratch_shapes=[
            pltpu.VMEM((2, LEAF_BLK, H), jnp.float32),
            pltpu.VMEM((N_LEAVES // 2, 2 * H), jnp.bfloat16),
            pltpu.VMEM((N_LEAVES // 4, 2 * H), jnp.bfloat16),
            pltpu.VMEM((N_LEAVES // 2, 2 * H), jnp.bfloat16),
            pltpu.VMEM((N_LEAVES // 4, 2 * H), jnp.bfloat16),
            pltpu.SemaphoreType.DMA((2,)),
        ],
    )(embeds, W_iou_w, W_iou_b, U_cat, b_cat, lin_w, lin_b)


def kernel(wordid, x, h, c, emb, W_iou_w, W_iou_b, U_iou_w, U_iou_b,
           U_f_w, U_f_b, lin_w, lin_b):
    del x, h, c

    embeds = _sc_gather(emb, wordid.astype(jnp.int32))

    W_iou_w = W_iou_w.astype(jnp.bfloat16)
    lin_w = lin_w.astype(jnp.bfloat16)

    U_cat = jnp.concatenate([U_f_w, U_iou_w],
                            axis=1).astype(jnp.bfloat16)
    b_cat = jnp.concatenate([U_f_b, U_iou_b]).reshape(1, -1)

    return _tree(embeds,
                 W_iou_w, W_iou_b.reshape(1, -1),
                 U_cat, b_cat,
                 lin_w, lin_b.reshape(1, -1))
